# baseline (speedup 1.0000x reference)
import jax
import jax.numpy as jnp
from jax import lax
from jax.experimental import pallas as pl
from jax.experimental.pallas import tpu as pltpu
from jax.experimental.pallas import tpu_sc as plsc

B, N, M, K = 8, 32768, 8192, 128
NC, NS, L = 2, 16, 16
NW = NC * NS
NT = N // 128
TSC = 64
TTC = NT - TSC
U = 8
QPT = TSC * 1024 // NW
GU = 8
CHUNKS = QPT // 128
PLANE_V = B * N
PLANE_C = B * M
PLANE_S = B * K * L


def _nn_body(verts_hbm, coll_hbm, cv_hbm, out_hbm,
             vxb, vyb, vzb, cvbuf, fidx, selb, srep, obuf, sem):
    wid = lax.axis_index("s") * NC + lax.axis_index("c")

    pltpu.sync_copy(cv_hbm, cvbuf)
    for c in range(3):
        dst = (vxb, vyb, vzb)[c]
        pltpu.sync_copy(verts_hbm.at[pl.ds(c * PLANE_V + wid * QPT, QPT)], dst)

    def fidx_body(r, carry):
        c = r // 8
        b = r - c * 8
        cb = c * PLANE_C + b * 128
        for jj in range(K // L):
            m = cvbuf[pl.ds(jj * L, L)]
            off = ((m >> 7) << 10) + (m & 127) + cb
            fidx[pl.ds(r * K + jj * L, L)] = off
        return carry

    lax.fori_loop(0, 24, fidx_body, 0)

    copies = [
        pltpu.async_copy(coll_hbm.at[fidx.at[pl.ds(r * K, K)]],
                         selb.at[pl.ds(r * K, K)], sem)
        for r in range(24)
    ]
    for cp in copies:
        cp.wait()

    def rep_body(r, carry):
        for jj in range(K // L):
            v16 = selb[pl.ds(r * K + jj * L, L)]
            for l in range(L):
                srep[pl.ds((r * K + jj * L + l) * L, L)] = (
                    jnp.full((L,), v16[l], jnp.float32))
        return carry

    lax.fori_loop(0, 24, rep_body, 0)

    inf = jnp.full((L,), jnp.inf, jnp.float32)
    zero = jnp.zeros((L,), jnp.int32)

    def chunk_body(c2, carry):
        b = c2 & 7
        p0 = c2 * 128
        vx, vy, vz = [], [], []
        for g in range(GU):
            vx.append(vxb[pl.ds(p0 + g * L, L)])
            vy.append(vyb[pl.ds(p0 + g * L, L)])
            vz.append(vzb[pl.ds(p0 + g * L, L)])

        sbase = b * (K * L)

        def k_body(k, bc):
            best, bidx = bc
            o = k * L
            kv = jnp.full((L,), k, jnp.int32)
            sx = srep[pl.ds(sbase + o, L)]
            sy = srep[pl.ds(PLANE_S + sbase + o, L)]
            sz = srep[pl.ds(2 * PLANE_S + sbase + o, L)]
            nbest, nbidx = [], []
            for g in range(GU):
                dx = vx[g] - sx
                dy = vy[g] - sy
                dz = vz[g] - sz
                d2 = (dx * dx + dy * dy) + dz * dz
                mlt = d2 < best[g]
                nbest.append(jnp.where(mlt, d2, best[g]))
                nbidx.append(jnp.where(mlt, kv, bidx[g]))
            return nbest, nbidx

        best, bidx = lax.fori_loop(0, K, k_body, ([inf] * GU, [zero] * GU),
                                   unroll=2)

        q = c2 >> 3
        obase = b * (CHUNKS // 8) * 256 + q * 256
        bvec = jnp.full((L,), b, jnp.int32)
        for g in range(GU):
            obuf[pl.ds(obase + g * L, L)] = bvec
            obuf[pl.ds(obase + 128 + g * L, L)] = bidx[g]
        return carry

    lax.fori_loop(0, CHUNKS, chunk_body, 0)

    qb = (CHUNKS // 8) * 256
    for b in range(B):
        pltpu.sync_copy(obuf.at[pl.ds(b * qb, qb)],
                        out_hbm.at[pl.ds(b * (TSC * 256) + wid * qb, qb)])


def _sc_half(vp, cp, cv):
    mesh = plsc.VectorSubcoreMesh(core_axis_name="c", subcore_axis_name="s")
    run = pl.kernel(
        _nn_body,
        out_type=jax.ShapeDtypeStruct((B * TSC * 256,), jnp.int32),
        mesh=mesh,
        compiler_params=pltpu.CompilerParams(needs_layout_passes=False),
        scratch_types=[
            pltpu.VMEM((QPT,), jnp.float32),
            pltpu.VMEM((QPT,), jnp.float32),
            pltpu.VMEM((QPT,), jnp.float32),
            pltpu.VMEM((K,), jnp.int32),
            pltpu.VMEM((24 * K,), jnp.int32),
            pltpu.VMEM((24 * K,), jnp.float32),
            pltpu.VMEM((3 * PLANE_S,), jnp.float32),
            pltpu.VMEM((B * CHUNKS // 8 * 256,), jnp.int32),
            pltpu.SemaphoreType.DMA,
        ],
    )
    return run(vp, cp, cv)


def _tc_body(vin, cin, cv, outr, sel3):
    @pl.when(pl.program_id(0) == 0)
    def _():
        def sel_body(k, carry):
            m = cv[k]
            tm = m >> 7
            lane = m & 127
            lmask = lax.broadcasted_iota(jnp.int32, (8, 128), 1) == lane
            for c in range(3):
                blk = cin[c, tm]
                v = jnp.sum(jnp.where(lmask, blk, 0.0), axis=1, keepdims=True)
                sel3[c, k] = jnp.broadcast_to(v, (8, 128))
            return carry

        lax.fori_loop(0, K, sel_body, 0)

    vx = [vin[0, u] for u in range(U)]
    vy = [vin[1, u] for u in range(U)]
    vz = [vin[2, u] for u in range(U)]
    inf = jnp.full((8, 128), jnp.inf, jnp.float32)
    zero = jnp.zeros((8, 128), jnp.int32)

    def k_body(k, bc):
        best, bidx = bc
        kv = jnp.full((8, 128), k, jnp.int32)
        sx = sel3[0, k]
        sy = sel3[1, k]
        sz = sel3[2, k]
        nbest, nbidx = [], []
        for u in range(U):
            dx = vx[u] - sx
            dy = vy[u] - sy
            dz = vz[u] - sz
            d2 = (dx * dx + dy * dy) + dz * dz
            mlt = d2 < best[u]
            nbest.append(jnp.where(mlt, d2, best[u]))
            nbidx.append(jnp.where(mlt, kv, bidx[u]))
        return nbest, nbidx

    best, bidx = lax.fori_loop(0, K, k_body, ([inf] * U, [zero] * U),
                               unroll=16)
    biota = lax.broadcasted_iota(jnp.int32, (8, 128), 0)
    for u in range(U):
        outr[:, u, 0, :] = biota
        outr[:, u, 1, :] = bidx[u]


def _tc_half(vp4, cp4, cv):
    return pl.pallas_call(
        _tc_body,
        grid=(TTC // U,),
        in_specs=[
            pl.BlockSpec((3, U, 8, 128), lambda t: (0, TSC // U + t, 0, 0)),
            pl.BlockSpec((3, M // 128, 8, 128), lambda t: (0, 0, 0, 0)),
            pl.BlockSpec(memory_space=pltpu.SMEM),
        ],
        out_specs=pl.BlockSpec((8, U, 2, 128), lambda t: (0, t, 0, 0)),
        out_shape=jax.ShapeDtypeStruct((B, TTC, 2, 128), jnp.int32),
        scratch_shapes=[pltpu.VMEM((3, K, 8, 128), jnp.float32)],
    )(vp4, cp4, cv)


def kernel(vertices, collider, collision_vertices):
    vp4 = jnp.transpose(
        jnp.transpose(vertices, (2, 0, 1)).reshape(3, B, N // 128, 128),
        (0, 2, 1, 3))
    vp = vp4.reshape(-1)
    cp4 = jnp.transpose(
        jnp.transpose(collider, (2, 0, 1)).reshape(3, B, M // 128, 128),
        (0, 2, 1, 3))
    cp = cp4.reshape(-1)

    x_sc = _sc_half(vp, cp, collision_vertices)
    x_tc = _tc_half(vp4, cp4, collision_vertices)

    x = jnp.concatenate([x_sc.reshape(B, TSC, 2, 128), x_tc], axis=1)
    return x.transpose(0, 1, 3, 2).reshape(B, N, 2)


# speedup vs baseline: 1.0647x; 1.0647x over previous
import jax
import jax.numpy as jnp
from jax import lax
from jax.experimental import pallas as pl
from jax.experimental.pallas import tpu as pltpu
from jax.experimental.pallas import tpu_sc as plsc

B, N, M, K = 8, 32768, 8192, 128
NC, NS, L = 2, 16, 16
NW = NC * NS
NT = N // 128
TSC = 64
TTC = NT - TSC
U = 8
QPT = TSC * 1024 // NW
GU = 8
CHUNKS = QPT // 128
PLANE_V = B * N
PLANE_C = B * M
PLANE_S = B * K * L


def _nn_body(verts_hbm, coll_hbm, cv_hbm, out_hbm,
             vxb, vyb, vzb, cvbuf, fidx, selb, srep, obuf, sem):
    wid = lax.axis_index("s") * NC + lax.axis_index("c")

    pltpu.sync_copy(cv_hbm, cvbuf)
    for c in range(3):
        dst = (vxb, vyb, vzb)[c]
        pltpu.sync_copy(verts_hbm.at[pl.ds(c * PLANE_V + wid * QPT, QPT)], dst)

    def fidx_body(r, carry):
        c = r // 8
        b = r - c * 8
        cb = c * PLANE_C + b * 128
        for jj in range(K // L):
            m = cvbuf[pl.ds(jj * L, L)]
            off = ((m >> 7) << 10) + (m & 127) + cb
            fidx[pl.ds(r * K + jj * L, L)] = off
        return carry

    lax.fori_loop(0, 24, fidx_body, 0)

    copies = [
        pltpu.async_copy(coll_hbm.at[fidx.at[pl.ds(r * K, K)]],
                         selb.at[pl.ds(r * K, K)], sem)
        for r in range(24)
    ]
    for cp in copies:
        cp.wait()

    def rep_body(r, carry):
        for jj in range(K // L):
            v16 = selb[pl.ds(r * K + jj * L, L)]
            for l in range(L):
                srep[pl.ds((r * K + jj * L + l) * L, L)] = (
                    jnp.full((L,), v16[l], jnp.float32))
        return carry

    lax.fori_loop(0, 24, rep_body, 0)

    inf = jnp.full((L,), jnp.inf, jnp.float32)
    zero = jnp.zeros((L,), jnp.int32)

    def chunk_body(c2, carry):
        b = c2 & 7
        p0 = c2 * 128
        vx, vy, vz = [], [], []
        for g in range(GU):
            vx.append(vxb[pl.ds(p0 + g * L, L)])
            vy.append(vyb[pl.ds(p0 + g * L, L)])
            vz.append(vzb[pl.ds(p0 + g * L, L)])

        sbase = b * (K * L)

        def k_body(k, bc):
            best, bidx = bc
            o = k * L
            kv = jnp.full((L,), k, jnp.int32)
            sx = srep[pl.ds(sbase + o, L)]
            sy = srep[pl.ds(PLANE_S + sbase + o, L)]
            sz = srep[pl.ds(2 * PLANE_S + sbase + o, L)]
            nbest, nbidx = [], []
            for g in range(GU):
                dx = vx[g] - sx
                dy = vy[g] - sy
                dz = vz[g] - sz
                d2 = (dx * dx + dy * dy) + dz * dz
                mlt = d2 < best[g]
                nbest.append(jnp.where(mlt, d2, best[g]))
                nbidx.append(jnp.where(mlt, kv, bidx[g]))
            return nbest, nbidx

        best, bidx = lax.fori_loop(0, K, k_body, ([inf] * GU, [zero] * GU))

        q = c2 >> 3
        obase = b * (CHUNKS // 8) * 256 + q * 256
        bvec = jnp.full((L,), b, jnp.int32)
        for g in range(GU):
            obuf[pl.ds(obase + g * L, L)] = bvec
            obuf[pl.ds(obase + 128 + g * L, L)] = bidx[g]
        return carry

    lax.fori_loop(0, CHUNKS, chunk_body, 0)

    qb = (CHUNKS // 8) * 256
    for b in range(B):
        pltpu.sync_copy(obuf.at[pl.ds(b * qb, qb)],
                        out_hbm.at[pl.ds(b * (TSC * 256) + wid * qb, qb)])


def _sc_half(vp, cp, cv):
    mesh = plsc.VectorSubcoreMesh(core_axis_name="c", subcore_axis_name="s")
    run = pl.kernel(
        _nn_body,
        out_type=jax.ShapeDtypeStruct((B * TSC * 256,), jnp.int32),
        mesh=mesh,
        compiler_params=pltpu.CompilerParams(needs_layout_passes=False),
        scratch_types=[
            pltpu.VMEM((QPT,), jnp.float32),
            pltpu.VMEM((QPT,), jnp.float32),
            pltpu.VMEM((QPT,), jnp.float32),
            pltpu.VMEM((K,), jnp.int32),
            pltpu.VMEM((24 * K,), jnp.int32),
            pltpu.VMEM((24 * K,), jnp.float32),
            pltpu.VMEM((3 * PLANE_S,), jnp.float32),
            pltpu.VMEM((B * CHUNKS // 8 * 256,), jnp.int32),
            pltpu.SemaphoreType.DMA,
        ],
    )
    return run(vp, cp, cv)


def _tc_body(vin, cin, cv, outr, sel3):
    @pl.when(pl.program_id(0) == 0)
    def _():
        def sel_body(k, carry):
            m = cv[k]
            tm = m >> 7
            lane = m & 127
            lmask = lax.broadcasted_iota(jnp.int32, (8, 128), 1) == lane
            for c in range(3):
                blk = cin[c, tm]
                v = jnp.sum(jnp.where(lmask, blk, 0.0), axis=1, keepdims=True)
                sel3[c, k] = jnp.broadcast_to(v, (8, 128))
            return carry

        lax.fori_loop(0, K, sel_body, 0)

    vx = [vin[0, u] for u in range(U)]
    vy = [vin[1, u] for u in range(U)]
    vz = [vin[2, u] for u in range(U)]
    inf = jnp.full((8, 128), jnp.inf, jnp.float32)
    zero = jnp.zeros((8, 128), jnp.int32)

    def k_body(k, bc):
        best, bidx = bc
        kv = jnp.full((8, 128), k, jnp.int32)
        sx = sel3[0, k]
        sy = sel3[1, k]
        sz = sel3[2, k]
        nbest, nbidx = [], []
        for u in range(U):
            dx = vx[u] - sx
            dy = vy[u] - sy
            dz = vz[u] - sz
            d2 = (dx * dx + dy * dy) + dz * dz
            mlt = d2 < best[u]
            nbest.append(jnp.where(mlt, d2, best[u]))
            nbidx.append(jnp.where(mlt, kv, bidx[u]))
        return nbest, nbidx

    best, bidx = lax.fori_loop(0, K, k_body, ([inf] * U, [zero] * U),
                               unroll=16)
    biota = lax.broadcasted_iota(jnp.int32, (8, 128), 0)
    for u in range(U):
        outr[:, u, 0, :] = biota
        outr[:, u, 1, :] = bidx[u]


def _tc_half(vp4, cp4, cv):
    return pl.pallas_call(
        _tc_body,
        grid=(TTC // U,),
        in_specs=[
            pl.BlockSpec((3, U, 8, 128), lambda t: (0, TSC // U + t, 0, 0)),
            pl.BlockSpec((3, M // 128, 8, 128), lambda t: (0, 0, 0, 0)),
            pl.BlockSpec(memory_space=pltpu.SMEM),
        ],
        out_specs=pl.BlockSpec((8, U, 2, 128), lambda t: (0, t, 0, 0)),
        out_shape=jax.ShapeDtypeStruct((B, TTC, 2, 128), jnp.int32),
        scratch_shapes=[pltpu.VMEM((3, K, 8, 128), jnp.float32)],
    )(vp4, cp4, cv)


def kernel(vertices, collider, collision_vertices):
    vp4 = jnp.transpose(
        jnp.transpose(vertices, (2, 0, 1)).reshape(3, B, N // 128, 128),
        (0, 2, 1, 3))
    vp = vp4.reshape(-1)
    cp4 = jnp.transpose(
        jnp.transpose(collider, (2, 0, 1)).reshape(3, B, M // 128, 128),
        (0, 2, 1, 3))
    cp = cp4.reshape(-1)

    x_sc = _sc_half(vp, cp, collision_vertices)
    x_tc = _tc_half(vp4, cp4, collision_vertices)

    x = jnp.concatenate([x_sc.reshape(B, TSC, 2, 128), x_tc], axis=1)
    return x.transpose(0, 1, 3, 2).reshape(B, N, 2)
